# Initial kernel scaffold; baseline (speedup 1.0000x reference)
import jax
import jax.numpy as jnp
from jax.experimental import pallas as pl


def kernel(speaker_memory, speaker_ids, edu_reps, W_ih, W_hh, b_ih, b_hh):
    raise NotImplementedError("write your pallas kernel here")


# speedup vs baseline: 1.4704x; 1.4704x over previous
import functools

import jax
import jax.numpy as jnp
from jax import lax
from jax.experimental import pallas as pl
from jax.experimental.pallas import tpu as pltpu
from jax.experimental.pallas import tpu_sc as plsc

M = 1000000
B = 16384
D = 64
NC = 2
NS = 16
NW = NC * NS
RPW = M // NW
CR = 125
NCHUNK = RPW // CR
SCH = 128
BPW = B // NW


def _wid():
    return lax.axis_index("s") * NC + lax.axis_index("c")


@functools.cache
def _make_sc_gather():
    return pl.kernel(
        _sc_gather_body,
        out_type=jax.ShapeDtypeStruct((B, D), jnp.float32),
        mesh=plsc.VectorSubcoreMesh(core_axis_name="c", subcore_axis_name="s"),
        compiler_params=pltpu.CompilerParams(use_tc_tiling_on_sc=False, needs_layout_passes=False),
        scratch_types=[
            pltpu.VMEM((BPW,), jnp.int32),
            pltpu.VMEM((BPW, D), jnp.float32),
            pltpu.SemaphoreType.DMA,
        ],
    )


def _sc_gather_body(mem_hbm, ids_hbm, out_hbm, idx_v, rows_v, sem):
    wid = _wid()
    base = wid * BPW
    pltpu.sync_copy(ids_hbm.at[pl.ds(base, BPW)], idx_v)
    pltpu.async_copy(mem_hbm.at[idx_v], rows_v, sem).wait()
    pltpu.sync_copy(rows_v, out_hbm.at[pl.ds(base, BPW)])


def _gru_body(x_ref, h_ref, wih_ref, whh_ref, bih_ref, bhh_ref, o_ref):
    x = x_ref[...]
    h = h_ref[...]
    gi = jnp.dot(x, wih_ref[...], preferred_element_type=jnp.float32) + bih_ref[...]
    gh = jnp.dot(h, whh_ref[...], preferred_element_type=jnp.float32) + bhh_ref[...]
    r = jax.nn.sigmoid(gi[:, 0:D] + gh[:, 0:D])
    z = jax.nn.sigmoid(gi[:, D:2 * D] + gh[:, D:2 * D])
    n = jnp.tanh(gi[:, 2 * D:3 * D] + r * gh[:, 2 * D:3 * D])
    o_ref[...] = (1.0 - z) * n + z * h


_GRU_BLK = 1024
_gru = pl.pallas_call(
    _gru_body,
    out_shape=jax.ShapeDtypeStruct((B, D), jnp.float32),
    grid=(B // _GRU_BLK,),
    in_specs=[
        pl.BlockSpec((_GRU_BLK, D), lambda i: (i, 0)),
        pl.BlockSpec((_GRU_BLK, D), lambda i: (i, 0)),
        pl.BlockSpec((D, 3 * D), lambda i: (0, 0)),
        pl.BlockSpec((D, 3 * D), lambda i: (0, 0)),
        pl.BlockSpec((1, 3 * D), lambda i: (0, 0)),
        pl.BlockSpec((1, 3 * D), lambda i: (0, 0)),
    ],
    out_specs=pl.BlockSpec((_GRU_BLK, D), lambda i: (i, 0)),
)


@functools.cache
def _make_sc_copy_scatter():
    return pl.kernel(
        _sc_copy_scatter_body,
        out_type=jax.ShapeDtypeStruct((M, D), jnp.float32),
        mesh=plsc.VectorSubcoreMesh(core_axis_name="c", subcore_axis_name="s"),
        compiler_params=pltpu.CompilerParams(use_tc_tiling_on_sc=False, needs_layout_passes=False),
        scratch_types=[
            pltpu.VMEM((2048,), jnp.int32),
            pltpu.VMEM((B,), jnp.int32),
            pltpu.VMEM((B,), jnp.int32),
            pltpu.VMEM((RPW,), jnp.int32),
            pltpu.VMEM((CR, D), jnp.float32),
            pltpu.VMEM((CR, D), jnp.float32),
            pltpu.VMEM((SCH,), jnp.int32),
            pltpu.VMEM((SCH,), jnp.int32),
            pltpu.VMEM((SCH, D), jnp.float32),
            pltpu.SemaphoreType.DMA,
            pltpu.SemaphoreType.DMA,
            pltpu.SemaphoreType.DMA,
            pltpu.SemaphoreType.DMA,
            pltpu.SemaphoreType.DMA,
        ],
    )


def _sc_copy_scatter_body(mem_hbm, ids_hbm, newh_hbm, out_hbm,
                     ids_loc, my_ids, my_pos, table, bufA, bufB,
                     id_chunk, pos_chunk, rows_sc,
                     sem_ia, sem_oa, sem_ib, sem_ob, sem_g):
    wid = _wid()
    base = wid * RPW

    def wait_dma(src, dst, sem):
        pltpu.make_async_copy(src, dst, sem).wait()

    iota = lax.broadcasted_iota(jnp.int32, (16,), 0)

    def outer(cb, n):
        pltpu.sync_copy(ids_hbm.at[pl.ds(cb * 2048, 2048)], ids_loc)

        def inner(i, n):
            v = ids_loc[pl.ds(i * 16, 16)]
            posv = cb * 2048 + i * 16 + iota
            msk = (v >= base) & (v < base + RPW)
            off = n + plsc.cumsum(msk.astype(jnp.int32)) - 1
            plsc.store_scatter(my_ids, [off], v, mask=msk)
            plsc.store_scatter(my_pos, [off], posv, mask=msk)
            return n + jnp.sum(msk.astype(jnp.int32))

        return lax.fori_loop(0, 2048 // 16, inner, n)

    n = lax.fori_loop(0, B // 2048, outer, 0)

    nv = (n + 15) // 16

    def dd1(j, _):
        s = j * 16
        idv = my_ids[pl.ds(s, 16)] - base
        jv = s + iota
        valid = jv < n
        for l in range(16):
            plsc.store_scatter(table, [idv], jv, mask=valid & (iota == l))
        return 0

    lax.fori_loop(0, nv, dd1, 0)

    def dd2(j, mcur):
        s = j * 16
        idv = my_ids[pl.ds(s, 16)]
        pv = my_pos[pl.ds(s, 16)]
        jv = s + iota
        valid = jv < n
        tv = plsc.load_gather(table, [jnp.clip(idv - base, 0, RPW - 1)])
        keep = valid & (tv == jv)
        off = mcur + plsc.cumsum(keep.astype(jnp.int32)) - 1
        plsc.store_scatter(my_ids, [off], idv, mask=keep)
        plsc.store_scatter(my_pos, [off], pv, mask=keep)
        return mcur + jnp.sum(keep.astype(jnp.int32))

    m = lax.fori_loop(0, nv, dd2, 0)

    mpad = ((m + SCH - 1) // SCH) * SCH
    zeros16 = jnp.zeros((16,), jnp.int32)

    @pl.when(m > 0)
    def _():
        id0 = plsc.load_gather(my_ids, [zeros16])
        pos0 = plsc.load_gather(my_pos, [zeros16])

        def padb(j, _):
            s = j * 16
            lane = s + iota
            cur_i = my_ids[pl.ds(s, 16)]
            cur_p = my_pos[pl.ds(s, 16)]
            my_ids[pl.ds(s, 16)] = jnp.where(lane >= m, id0, cur_i)
            my_pos[pl.ds(s, 16)] = jnp.where(lane >= m, pos0, cur_p)
            return 0

        lax.fori_loop(m // 16, mpad // 16, padb, 0)

    pltpu.async_copy(mem_hbm.at[pl.ds(base, CR)], bufA, sem_ia)
    pltpu.async_copy(mem_hbm.at[pl.ds(base + CR, CR)], bufB, sem_ib)

    def cpb(t, _):
        cA = 2 * t
        cB = 2 * t + 1
        rA = base + cA * CR
        rB = base + cB * CR
        wait_dma(mem_hbm.at[pl.ds(rA, CR)], bufA, sem_ia)
        pltpu.async_copy(bufA, out_hbm.at[pl.ds(rA, CR)], sem_oa)
        wait_dma(mem_hbm.at[pl.ds(rB, CR)], bufB, sem_ib)
        pltpu.async_copy(bufB, out_hbm.at[pl.ds(rB, CR)], sem_ob)
        wait_dma(bufA, out_hbm.at[pl.ds(rA, CR)], sem_oa)
        pltpu.async_copy(mem_hbm.at[pl.ds(rA + 2 * CR, CR)], bufA, sem_ia)
        wait_dma(bufB, out_hbm.at[pl.ds(rB, CR)], sem_ob)
        pltpu.async_copy(mem_hbm.at[pl.ds(rB + 2 * CR, CR)], bufB, sem_ib)
        return 0

    lax.fori_loop(0, NCHUNK // 2 - 1, cpb, 0)
    cA = NCHUNK - 2
    cB = NCHUNK - 1
    rA = base + cA * CR
    rB = base + cB * CR
    wait_dma(mem_hbm.at[pl.ds(rA, CR)], bufA, sem_ia)
    pltpu.async_copy(bufA, out_hbm.at[pl.ds(rA, CR)], sem_oa)
    wait_dma(mem_hbm.at[pl.ds(rB, CR)], bufB, sem_ib)
    pltpu.async_copy(bufB, out_hbm.at[pl.ds(rB, CR)], sem_ob)
    wait_dma(bufA, out_hbm.at[pl.ds(rA, CR)], sem_oa)
    wait_dma(bufB, out_hbm.at[pl.ds(rB, CR)], sem_ob)

    nsc = (m + SCH - 1) // SCH

    def scb(k, _):
        s = k * SCH
        for l in range(SCH // 16):
            id_chunk[pl.ds(l * 16, 16)] = my_ids[pl.ds(s + l * 16, 16)]
            pos_chunk[pl.ds(l * 16, 16)] = my_pos[pl.ds(s + l * 16, 16)]
        pltpu.async_copy(newh_hbm.at[pos_chunk], rows_sc, sem_g).wait()
        pltpu.async_copy(rows_sc, out_hbm.at[id_chunk], sem_g).wait()
        return 0

    lax.fori_loop(0, nsc, scb, 0)


def kernel(speaker_memory, speaker_ids, edu_reps, W_ih, W_hh, b_ih, b_hh):
    ids32 = speaker_ids.astype(jnp.int32)
    h = _make_sc_gather()(speaker_memory, ids32)
    new_h = _gru(edu_reps, h, W_ih.T, W_hh.T,
                 b_ih.reshape(1, 3 * D), b_hh.reshape(1, 3 * D))
    return _make_sc_copy_scatter()(speaker_memory, ids32, new_h)
